# Initial kernel scaffold; baseline (speedup 1.0000x reference)
import jax
import jax.numpy as jnp
from jax.experimental import pallas as pl


def kernel(x, embed_weight):
    raise NotImplementedError("write your pallas kernel here")


# speedup vs baseline: 2.7330x; 2.7330x over previous
import jax
import jax.numpy as jnp
from jax.experimental import pallas as pl


def _copy_body(w_ref, o_ref):
    o_ref[...] = w_ref[...]


def kernel(x, embed_weight):
    seq_len = x.shape[1]
    n_model = embed_weight.shape[1]
    block_rows = 512
    while seq_len % block_rows:
        block_rows //= 2
    grid = (seq_len // block_rows,)
    return pl.pallas_call(
        _copy_body,
        grid=grid,
        in_specs=[pl.BlockSpec((block_rows, n_model), lambda i: (i, 0))],
        out_specs=pl.BlockSpec((block_rows, n_model), lambda i: (i, 0)),
        out_shape=jax.ShapeDtypeStruct((seq_len, n_model), embed_weight.dtype),
    )(embed_weight)


# speedup vs baseline: 3.2292x; 1.1816x over previous
import jax
import jax.numpy as jnp
from jax.experimental import pallas as pl


def _copy_body(w_ref, o_ref):
    o_ref[...] = w_ref[...]


def kernel(x, embed_weight):
    seq_len = x.shape[1]
    n_model = embed_weight.shape[1]
    block_rows = 2048
    while seq_len % block_rows:
        block_rows //= 2
    grid = (seq_len // block_rows,)
    return pl.pallas_call(
        _copy_body,
        grid=grid,
        in_specs=[pl.BlockSpec((block_rows, n_model), lambda i: (i, 0))],
        out_specs=pl.BlockSpec((block_rows, n_model), lambda i: (i, 0)),
        out_shape=jax.ShapeDtypeStruct((seq_len, n_model), embed_weight.dtype),
    )(embed_weight)
